# Initial kernel scaffold; baseline (speedup 1.0000x reference)
import jax
import jax.numpy as jnp
from jax.experimental import pallas as pl


def kernel(x, edge_index, edge_weight, edge_attr, nn_w1, nn_b1, nn_w2, nn_b2, lin1_w, lin2_w, lin2_b, lin_w, lin_b):
    raise NotImplementedError("write your pallas kernel here")


# speedup vs baseline: 1.2119x; 1.2119x over previous
import functools

import jax
import jax.numpy as jnp
from jax import lax
from jax.experimental import pallas as pl
from jax.experimental.pallas import tpu as pltpu
from jax.experimental.pallas import tpu_sc as plsc

CUTOFF = 10.0

NC = 2
NS = 16
LANES = 16
CHUNK = 128
GROUP = 8


def _ssp(v):
    return jax.nn.softplus(v) - jnp.log(2.0)


def _h_body(x_ref, w_ref, out_ref):
    xb = x_ref[...]
    w = w_ref[...]
    h = lax.dot_general(xb, w, (((1,), (1,)), ((), ())),
                        preferred_element_type=jnp.float32)
    half = w.shape[0] // 2
    out_ref[0] = h[:, :half]
    out_ref[1] = h[:, half:]


def _compute_h_slabs(x, lin1_w, bn):
    n, f = x.shape
    half = f // 2
    return pl.pallas_call(
        _h_body,
        grid=(n // bn,),
        in_specs=[
            pl.BlockSpec((bn, f), lambda i: (i, 0)),
            pl.BlockSpec((f, f), lambda i: (0, 0)),
        ],
        out_specs=pl.BlockSpec((2, bn, half), lambda i: (0, i, 0)),
        out_shape=jax.ShapeDtypeStruct((2, n, half), jnp.float32),
    )(x, lin1_w)


def _w_body(e_real, be, ea_ref, ew_ref, w1_ref, b1_ref, w2_ref, b2_ref,
            out_ref):
    i = pl.program_id(0)
    ea = ea_ref[...]
    u = lax.dot_general(ea, w1_ref[...], (((1,), (1,)), ((), ())),
                        preferred_element_type=jnp.float32)
    u = _ssp(u + b1_ref[...])
    w = lax.dot_general(u, w2_ref[...], (((1,), (1,)), ((), ())),
                        preferred_element_type=jnp.float32)
    w = w + b2_ref[...]
    c = 0.5 * (jnp.cos(ew_ref[...] * (jnp.pi / CUTOFF)) + 1.0)
    w = w * c
    rows = lax.broadcasted_iota(jnp.int32, w.shape, 0) + i * be
    w = jnp.where(rows < e_real, w, 0.0)
    half = w.shape[1] // 2
    out_ref[0] = w[:, :half]
    out_ref[1] = w[:, half:]


def _compute_w_slabs(ea_pad, ew_pad, w1, b1, w2, b2, e_real, be):
    e_pad, r = ea_pad.shape
    f = w2.shape[0]
    half = f // 2
    return pl.pallas_call(
        functools.partial(_w_body, e_real, be),
        grid=(e_pad // be,),
        in_specs=[
            pl.BlockSpec((be, r), lambda i: (i, 0)),
            pl.BlockSpec((be, 1), lambda i: (i, 0)),
            pl.BlockSpec((f, r), lambda i: (0, 0)),
            pl.BlockSpec((1, f), lambda i: (0, 0)),
            pl.BlockSpec((f, f), lambda i: (0, 0)),
            pl.BlockSpec((1, f), lambda i: (0, 0)),
        ],
        out_specs=pl.BlockSpec((2, be, half), lambda i: (0, i, 0)),
        out_shape=jax.ShapeDtypeStruct((2, e_pad, half), jnp.float32),
    )(ea_pad, ew_pad, w1, b1.reshape(1, f), w2, b2.reshape(1, f))


def _sc_body(n_nodes, stripe, chunks, half, h_hbm, w_hbm, src_hbm, dst_hbm,
             z_hbm, out_hbm, acc_sh, sbuf, dbuf, wbuf, hbuf):
    c = lax.axis_index("c")
    s = lax.axis_index("s")
    cs = c * NS + s
    vregs = half // LANES

    stripe0 = pl.multiple_of(s * stripe, 8)
    pltpu.sync_copy(z_hbm, acc_sh.at[pl.ds(stripe0, stripe)])
    plsc.subcore_barrier()

    def _group(g0, _):
        grow = pl.multiple_of(g0 * GROUP, 8)
        pltpu.sync_copy(src_hbm.at[c, s].at[pl.ds(grow, GROUP)], sbuf)
        pltpu.sync_copy(dst_hbm.at[s].at[pl.ds(grow, GROUP)], dbuf)
        for j in range(GROUP):
            pltpu.sync_copy(w_hbm.at[cs, grow + j], wbuf)
            pltpu.sync_copy(h_hbm.at[sbuf.at[j]], hbuf)

            def _mul(i, _):
                for v in range(vregs):
                    sl = pl.ds(v * LANES, LANES)
                    wbuf[i, sl] = wbuf[i, sl] * hbuf[i, sl]
                return 0

            lax.fori_loop(0, CHUNK, _mul, 0)
            pltpu.sync_copy(wbuf, acc_sh.at[dbuf.at[j]], add=True)
        return 0

    lax.fori_loop(0, chunks // GROUP, _group, 0)
    plsc.subcore_barrier()

    pltpu.sync_copy(acc_sh.at[pl.ds(stripe0, stripe)], out_hbm.at[cs])


def _sc_aggregate(hflat, w4d, src4d, dst3d, n_nodes, e_pad, half):
    mesh = plsc.VectorSubcoreMesh(core_axis_name="c", subcore_axis_name="s")
    chunks = e_pad // NS // CHUNK
    stripe = ((n_nodes + NS - 1) // NS + 7) // 8 * 8
    npad = NS * stripe
    run = pl.kernel(
        functools.partial(_sc_body, n_nodes, stripe, chunks, half),
        out_type=jax.ShapeDtypeStruct((NC * NS, stripe, half), jnp.float32),
        mesh=mesh,
        scratch_types=[
            pltpu.VMEM_SHARED((npad, half), jnp.float32),
            pltpu.VMEM((GROUP, CHUNK), jnp.int32),
            pltpu.VMEM((GROUP, CHUNK), jnp.int32),
            pltpu.VMEM((CHUNK, half), jnp.float32),
            pltpu.VMEM((CHUNK, half), jnp.float32),
        ],
    )
    zeros = jnp.zeros((stripe, half), jnp.float32)
    out = run(hflat, w4d, src4d, dst3d, zeros)
    return out.reshape(NC, NS * stripe, half)[:, :n_nodes, :]


def _tail_body(agg_ref, l2w_ref, l2b_ref, lw_ref, lb_ref, out_ref):
    a0 = agg_ref[0]
    a1 = agg_ref[1]
    l2w = l2w_ref[...]
    half = a0.shape[1]
    conv = lax.dot_general(a0, l2w[:, :half], (((1,), (1,)), ((), ())),
                           preferred_element_type=jnp.float32)
    conv = conv + lax.dot_general(a1, l2w[:, half:],
                                  (((1,), (1,)), ((), ())),
                                  preferred_element_type=jnp.float32)
    t = jnp.tanh(conv + l2b_ref[...])
    out = lax.dot_general(t, lw_ref[...], (((1,), (1,)), ((), ())),
                          preferred_element_type=jnp.float32)
    out_ref[...] = out + lb_ref[...]


def _tail(agg_slabs, lin2_w, lin2_b, lin_w, lin_b, bn):
    _, n, half = agg_slabs.shape
    f = lin2_w.shape[0]
    return pl.pallas_call(
        _tail_body,
        grid=(n // bn,),
        in_specs=[
            pl.BlockSpec((2, bn, half), lambda i: (0, i, 0)),
            pl.BlockSpec((f, f), lambda i: (0, 0)),
            pl.BlockSpec((1, f), lambda i: (0, 0)),
            pl.BlockSpec((f, f), lambda i: (0, 0)),
            pl.BlockSpec((1, f), lambda i: (0, 0)),
        ],
        out_specs=pl.BlockSpec((bn, f), lambda i: (i, 0)),
        out_shape=jax.ShapeDtypeStruct((n, f), jnp.float32),
    )(agg_slabs, lin2_w, lin2_b.reshape(1, f), lin_w, lin_b.reshape(1, f))


def kernel(x, edge_index, edge_weight, edge_attr, nn_w1, nn_b1, nn_w2,
           nn_b2, lin1_w, lin2_w, lin2_b, lin_w, lin_b):
    n, f = x.shape
    e = edge_index.shape[1]
    half = f // 2

    be = 2048
    grain = max(NS * CHUNK * GROUP, be)
    e_pad = ((e + grain - 1) // grain) * grain
    pad = e_pad - e

    src = edge_index[0]
    dst = edge_index[1]
    if pad:
        zi = jnp.zeros((pad,), jnp.int32)
        src = jnp.concatenate([src, zi])
        dst = jnp.concatenate([dst, zi])
        edge_attr = jnp.concatenate(
            [edge_attr, jnp.zeros((pad, edge_attr.shape[1]), jnp.float32)])
        edge_weight = jnp.concatenate(
            [edge_weight, jnp.zeros((pad,), jnp.float32)])

    h_slabs = _compute_h_slabs(x, lin1_w, bn=1000)
    w_slabs = _compute_w_slabs(edge_attr, edge_weight.reshape(e_pad, 1),
                               nn_w1, nn_b1, nn_w2, nn_b2, e, be=be)

    chunks = e_pad // NS // CHUNK
    hflat = h_slabs.reshape(NC * n, half)
    w4d = w_slabs.reshape(NC * NS, chunks, CHUNK, half)
    src4d = jnp.stack([src, src + n]).reshape(NC, NS, chunks, CHUNK)
    dst3d = dst.reshape(NS, chunks, CHUNK)

    agg_slabs = _sc_aggregate(hflat, w4d, src4d, dst3d, n, e_pad, half)

    return _tail(agg_slabs, lin2_w, lin2_b, lin_w, lin_b, bn=1000)


# speedup vs baseline: 1.2188x; 1.0057x over previous
import functools

import jax
import jax.numpy as jnp
from jax import lax
from jax.experimental import pallas as pl
from jax.experimental.pallas import tpu as pltpu
from jax.experimental.pallas import tpu_sc as plsc

CUTOFF = 10.0

NC = 2
NS = 16
LANES = 16
CHUNK = 128
GROUP = 8


def _ssp(v):
    return jax.nn.softplus(v) - jnp.log(2.0)


def _h_body(x_ref, w_ref, out_ref):
    xb = x_ref[...]
    w = w_ref[...]
    h = lax.dot_general(xb, w, (((1,), (1,)), ((), ())),
                        preferred_element_type=jnp.float32)
    half = w.shape[0] // 2
    out_ref[0] = h[:, :half]
    out_ref[1] = h[:, half:]


def _compute_h_slabs(x, lin1_w, bn):
    n, f = x.shape
    half = f // 2
    return pl.pallas_call(
        _h_body,
        grid=(n // bn,),
        in_specs=[
            pl.BlockSpec((bn, f), lambda i: (i, 0)),
            pl.BlockSpec((f, f), lambda i: (0, 0)),
        ],
        out_specs=pl.BlockSpec((2, bn, half), lambda i: (0, i, 0)),
        out_shape=jax.ShapeDtypeStruct((2, n, half), jnp.float32),
    )(x, lin1_w)


def _w_body(ea_ref, ew_ref, w1_ref, b1_ref, w2_ref, b2_ref, out_ref):
    ea = ea_ref[...]
    u = lax.dot_general(ea, w1_ref[...], (((1,), (1,)), ((), ())),
                        preferred_element_type=jnp.float32)
    u = _ssp(u + b1_ref[...])
    w = lax.dot_general(u, w2_ref[...], (((1,), (1,)), ((), ())),
                        preferred_element_type=jnp.float32)
    w = w + b2_ref[...]
    c = 0.5 * (jnp.cos(ew_ref[...] * (jnp.pi / CUTOFF)) + 1.0)
    w = w * c
    half = w.shape[1] // 2
    out_ref[0] = w[:, :half]
    out_ref[1] = w[:, half:]


def _compute_w_slabs(ea, ew, w1, b1, w2, b2, e_pad, be):
    e, r = ea.shape
    f = w2.shape[0]
    half = f // 2
    last = (e + be - 1) // be - 1
    return pl.pallas_call(
        _w_body,
        grid=(e_pad // be,),
        in_specs=[
            pl.BlockSpec((be, r), lambda i: (jnp.minimum(i, last), 0)),
            pl.BlockSpec((be, 1), lambda i: (jnp.minimum(i, last), 0)),
            pl.BlockSpec((f, r), lambda i: (0, 0)),
            pl.BlockSpec((1, f), lambda i: (0, 0)),
            pl.BlockSpec((f, f), lambda i: (0, 0)),
            pl.BlockSpec((1, f), lambda i: (0, 0)),
        ],
        out_specs=pl.BlockSpec((2, be, half), lambda i: (0, i, 0)),
        out_shape=jax.ShapeDtypeStruct((2, e_pad, half), jnp.float32),
    )(ea, ew.reshape(e, 1), w1, b1.reshape(1, f), w2, b2.reshape(1, f))


def _sc_body(n_nodes, stripe, chunks, half, h_hbm, w_hbm, src_hbm, dst_hbm,
             z_hbm, out_hbm, acc_sh, sbuf, dbuf, wbuf, hbuf):
    c = lax.axis_index("c")
    s = lax.axis_index("s")
    cs = c * NS + s
    vregs = half // LANES

    stripe0 = pl.multiple_of(s * stripe, 8)
    pltpu.sync_copy(z_hbm, acc_sh.at[pl.ds(stripe0, stripe)])
    plsc.subcore_barrier()

    def _group(g0, _):
        grow = pl.multiple_of(g0 * GROUP, 8)
        pltpu.sync_copy(src_hbm.at[c, s].at[pl.ds(grow, GROUP)], sbuf)
        pltpu.sync_copy(dst_hbm.at[s].at[pl.ds(grow, GROUP)], dbuf)
        for j in range(GROUP):
            pltpu.sync_copy(w_hbm.at[cs, grow + j], wbuf)
            pltpu.sync_copy(h_hbm.at[sbuf.at[j]], hbuf)

            def _mul(i, _):
                for v in range(vregs):
                    sl = pl.ds(v * LANES, LANES)
                    wbuf[i, sl] = wbuf[i, sl] * hbuf[i, sl]
                return 0

            lax.fori_loop(0, CHUNK, _mul, 0)
            pltpu.sync_copy(wbuf, acc_sh.at[dbuf.at[j]], add=True)
        return 0

    lax.fori_loop(0, chunks // GROUP, _group, 0)
    plsc.subcore_barrier()

    pltpu.sync_copy(acc_sh.at[pl.ds(stripe0, stripe)], out_hbm.at[cs])


def _sc_aggregate(hflat, w4d, src4d, dst3d, n_nodes, e_pad, half):
    mesh = plsc.VectorSubcoreMesh(core_axis_name="c", subcore_axis_name="s")
    chunks = e_pad // NS // CHUNK
    stripe = ((n_nodes + NS - 1) // NS + 7) // 8 * 8
    if NS * stripe <= n_nodes:
        stripe += 8
    npad = NS * stripe
    run = pl.kernel(
        functools.partial(_sc_body, n_nodes, stripe, chunks, half),
        out_type=jax.ShapeDtypeStruct((NC * NS, stripe, half), jnp.float32),
        mesh=mesh,
        scratch_types=[
            pltpu.VMEM_SHARED((npad, half), jnp.float32),
            pltpu.VMEM((GROUP, CHUNK), jnp.int32),
            pltpu.VMEM((GROUP, CHUNK), jnp.int32),
            pltpu.VMEM((CHUNK, half), jnp.float32),
            pltpu.VMEM((CHUNK, half), jnp.float32),
        ],
    )
    zeros = jnp.zeros((stripe, half), jnp.float32)
    out = run(hflat, w4d, src4d, dst3d, zeros)
    return out.reshape(NC, NS * stripe, half)


def _tail_body(agg_ref, l2w_ref, l2b_ref, lw_ref, lb_ref, out_ref):
    a0 = agg_ref[0]
    a1 = agg_ref[1]
    l2w = l2w_ref[...]
    half = a0.shape[1]
    conv = lax.dot_general(a0, l2w[:, :half], (((1,), (1,)), ((), ())),
                           preferred_element_type=jnp.float32)
    conv = conv + lax.dot_general(a1, l2w[:, half:],
                                  (((1,), (1,)), ((), ())),
                                  preferred_element_type=jnp.float32)
    t = jnp.tanh(conv + l2b_ref[...])
    out = lax.dot_general(t, lw_ref[...], (((1,), (1,)), ((), ())),
                          preferred_element_type=jnp.float32)
    out_ref[...] = out + lb_ref[...]


def _tail(agg_slabs, lin2_w, lin2_b, lin_w, lin_b, n, bn):
    half = agg_slabs.shape[2]
    f = lin2_w.shape[0]
    return pl.pallas_call(
        _tail_body,
        grid=(n // bn,),
        in_specs=[
            pl.BlockSpec((2, bn, half), lambda i: (0, i, 0)),
            pl.BlockSpec((f, f), lambda i: (0, 0)),
            pl.BlockSpec((1, f), lambda i: (0, 0)),
            pl.BlockSpec((f, f), lambda i: (0, 0)),
            pl.BlockSpec((1, f), lambda i: (0, 0)),
        ],
        out_specs=pl.BlockSpec((bn, f), lambda i: (i, 0)),
        out_shape=jax.ShapeDtypeStruct((n, f), jnp.float32),
    )(agg_slabs, lin2_w, lin2_b.reshape(1, f), lin_w, lin_b.reshape(1, f))


def kernel(x, edge_index, edge_weight, edge_attr, nn_w1, nn_b1, nn_w2,
           nn_b2, lin1_w, lin2_w, lin2_b, lin_w, lin_b):
    n, f = x.shape
    e = edge_index.shape[1]
    half = f // 2

    be = 2048
    grain = max(NS * CHUNK * GROUP, be)
    e_pad = ((e + grain - 1) // grain) * grain
    pad = e_pad - e

    src = edge_index[0]
    dst = edge_index[1]
    if pad:
        zi = jnp.zeros((pad,), jnp.int32)
        src = jnp.concatenate([src, zi])
        dst = jnp.concatenate([dst, jnp.full((pad,), n, jnp.int32)])

    h_slabs = _compute_h_slabs(x, lin1_w, bn=1000)
    w_slabs = _compute_w_slabs(edge_attr, edge_weight, nn_w1, nn_b1,
                               nn_w2, nn_b2, e_pad, be=be)

    chunks = e_pad // NS // CHUNK
    hflat = h_slabs.reshape(NC * n, half)
    w4d = w_slabs.reshape(NC * NS, chunks, CHUNK, half)
    src4d = jnp.stack([src, src + n]).reshape(NC, NS, chunks, CHUNK)
    dst3d = dst.reshape(NS, chunks, CHUNK)

    agg_slabs = _sc_aggregate(hflat, w4d, src4d, dst3d, n, e_pad, half)

    return _tail(agg_slabs, lin2_w, lin2_b, lin_w, lin_b, n, bn=1000)


# speedup vs baseline: 1.3948x; 1.1444x over previous
import functools

import jax
import jax.numpy as jnp
from jax import lax
from jax.experimental import pallas as pl
from jax.experimental.pallas import tpu as pltpu
from jax.experimental.pallas import tpu_sc as plsc

CUTOFF = 10.0

NC = 2
NS = 16
LANES = 16
CHUNK = 64
BODY = 8


def _ssp(v):
    return jax.nn.softplus(v) - jnp.log(2.0)


def _h_body(x_ref, w_ref, out_ref):
    xb = x_ref[...]
    w = w_ref[...]
    h = lax.dot_general(xb, w, (((1,), (1,)), ((), ())),
                        preferred_element_type=jnp.float32)
    half = w.shape[0] // 2
    out_ref[0] = h[:, :half]
    out_ref[1] = h[:, half:]


def _compute_h_slabs(x, lin1_w, bn):
    n, f = x.shape
    half = f // 2
    return pl.pallas_call(
        _h_body,
        grid=(n // bn,),
        in_specs=[
            pl.BlockSpec((bn, f), lambda i: (i, 0)),
            pl.BlockSpec((f, f), lambda i: (0, 0)),
        ],
        out_specs=pl.BlockSpec((2, bn, half), lambda i: (0, i, 0)),
        out_shape=jax.ShapeDtypeStruct((2, n, half), jnp.float32),
    )(x, lin1_w)


def _w_body(ea_ref, ew_ref, w1_ref, b1_ref, w2_ref, b2_ref, out_ref):
    ea = ea_ref[...]
    u = lax.dot_general(ea, w1_ref[...], (((1,), (1,)), ((), ())),
                        preferred_element_type=jnp.float32)
    u = _ssp(u + b1_ref[...])
    w = lax.dot_general(u, w2_ref[...], (((1,), (1,)), ((), ())),
                        preferred_element_type=jnp.float32)
    w = w + b2_ref[...]
    c = 0.5 * (jnp.cos(ew_ref[...] * (jnp.pi / CUTOFF)) + 1.0)
    w = w * c
    half = w.shape[1] // 2
    out_ref[0] = w[:, :half]
    out_ref[1] = w[:, half:]


def _compute_w_slabs(ea, ew, w1, b1, w2, b2, e_pad, be):
    e, r = ea.shape
    f = w2.shape[0]
    half = f // 2
    last = (e + be - 1) // be - 1
    return pl.pallas_call(
        _w_body,
        grid=(e_pad // be,),
        in_specs=[
            pl.BlockSpec((be, r), lambda i: (jnp.minimum(i, last), 0)),
            pl.BlockSpec((be, 1), lambda i: (jnp.minimum(i, last), 0)),
            pl.BlockSpec((f, r), lambda i: (0, 0)),
            pl.BlockSpec((1, f), lambda i: (0, 0)),
            pl.BlockSpec((f, f), lambda i: (0, 0)),
            pl.BlockSpec((1, f), lambda i: (0, 0)),
        ],
        out_specs=pl.BlockSpec((2, be, half), lambda i: (0, i, 0)),
        out_shape=jax.ShapeDtypeStruct((2, e_pad, half), jnp.float32),
    )(ea, ew.reshape(e, 1), w1, b1.reshape(1, f), w2, b2.reshape(1, f))


def _sc_body(n_nodes, stripe, chunks, half, h_hbm, w_hbm, src_hbm, dst_hbm,
             z_hbm, out_hbm, acc_sh, sbuf, dbuf, wbuf0, wbuf1, hbuf0,
             hbuf1, sem_w0, sem_w1, sem_h0, sem_h1, sem_s0, sem_s1):
    c = lax.axis_index("c")
    s = lax.axis_index("s")
    cs = c * NS + s
    vregs = half // LANES

    stripe0 = pl.multiple_of(s * stripe, 8)
    pltpu.sync_copy(z_hbm, acc_sh.at[pl.ds(stripe0, stripe)])
    plsc.subcore_barrier()

    wb = (wbuf0, wbuf1)
    hb = (hbuf0, hbuf1)
    sw = (sem_w0, sem_w1)
    sh = (sem_h0, sem_h1)
    ss = (sem_s0, sem_s1)

    def _body(u, _):
        base = pl.multiple_of(u * BODY, 8)
        pltpu.sync_copy(src_hbm.at[c, s].at[pl.ds(base, BODY)], sbuf)
        pltpu.sync_copy(dst_hbm.at[s].at[pl.ds(base, BODY)], dbuf)
        loads = {}
        scats = {}
        loads[0] = (
            pltpu.async_copy(w_hbm.at[cs, base], wb[0], sw[0]),
            pltpu.async_copy(h_hbm.at[sbuf.at[0]], hb[0], sh[0]),
        )
        for j in range(BODY):
            p = j & 1
            q = 1 - p
            if j + 1 < BODY:
                if j >= 1:
                    scats[j - 1].wait()
                loads[j + 1] = (
                    pltpu.async_copy(w_hbm.at[cs, base + j + 1],
                                     wb[q], sw[q]),
                    pltpu.async_copy(h_hbm.at[sbuf.at[j + 1]],
                                     hb[q], sh[q]),
                )
            loads[j][0].wait()
            loads[j][1].wait()
            wp, hp = wb[p], hb[p]

            def _mul(i, _):
                for v in range(vregs):
                    sl = pl.ds(v * LANES, LANES)
                    wp[i, sl] = wp[i, sl] * hp[i, sl]
                return 0

            lax.fori_loop(0, CHUNK, _mul, 0)
            scats[j] = pltpu.async_copy(wp, acc_sh.at[dbuf.at[j]],
                                        ss[p], add=True)
        scats[BODY - 2].wait()
        scats[BODY - 1].wait()
        return 0

    lax.fori_loop(0, chunks // BODY, _body, 0)
    plsc.subcore_barrier()

    pltpu.sync_copy(acc_sh.at[pl.ds(stripe0, stripe)], out_hbm.at[cs])


def _sc_aggregate(hflat, w4d, src4d, dst3d, n_nodes, e_pad, half):
    mesh = plsc.VectorSubcoreMesh(core_axis_name="c", subcore_axis_name="s")
    chunks = e_pad // NS // CHUNK
    stripe = ((n_nodes + NS - 1) // NS + 7) // 8 * 8
    if NS * stripe <= n_nodes:
        stripe += 8
    npad = NS * stripe
    run = pl.kernel(
        functools.partial(_sc_body, n_nodes, stripe, chunks, half),
        out_type=jax.ShapeDtypeStruct((NC * NS, stripe, half), jnp.float32),
        mesh=mesh,
        scratch_types=[
            pltpu.VMEM_SHARED((npad, half), jnp.float32),
            pltpu.VMEM((BODY, CHUNK), jnp.int32),
            pltpu.VMEM((BODY, CHUNK), jnp.int32),
            pltpu.VMEM((CHUNK, half), jnp.float32),
            pltpu.VMEM((CHUNK, half), jnp.float32),
            pltpu.VMEM((CHUNK, half), jnp.float32),
            pltpu.VMEM((CHUNK, half), jnp.float32),
            pltpu.SemaphoreType.DMA,
            pltpu.SemaphoreType.DMA,
            pltpu.SemaphoreType.DMA,
            pltpu.SemaphoreType.DMA,
            pltpu.SemaphoreType.DMA,
            pltpu.SemaphoreType.DMA,
        ],
    )
    zeros = jnp.zeros((stripe, half), jnp.float32)
    out = run(hflat, w4d, src4d, dst3d, zeros)
    return out.reshape(NC, NS * stripe, half)


def _tail_body(agg_ref, l2w_ref, l2b_ref, lw_ref, lb_ref, out_ref):
    a0 = agg_ref[0]
    a1 = agg_ref[1]
    l2w = l2w_ref[...]
    half = a0.shape[1]
    conv = lax.dot_general(a0, l2w[:, :half], (((1,), (1,)), ((), ())),
                           preferred_element_type=jnp.float32)
    conv = conv + lax.dot_general(a1, l2w[:, half:],
                                  (((1,), (1,)), ((), ())),
                                  preferred_element_type=jnp.float32)
    t = jnp.tanh(conv + l2b_ref[...])
    out = lax.dot_general(t, lw_ref[...], (((1,), (1,)), ((), ())),
                          preferred_element_type=jnp.float32)
    out_ref[...] = out + lb_ref[...]


def _tail(agg_slabs, lin2_w, lin2_b, lin_w, lin_b, n, bn):
    half = agg_slabs.shape[2]
    f = lin2_w.shape[0]
    return pl.pallas_call(
        _tail_body,
        grid=(n // bn,),
        in_specs=[
            pl.BlockSpec((2, bn, half), lambda i: (0, i, 0)),
            pl.BlockSpec((f, f), lambda i: (0, 0)),
            pl.BlockSpec((1, f), lambda i: (0, 0)),
            pl.BlockSpec((f, f), lambda i: (0, 0)),
            pl.BlockSpec((1, f), lambda i: (0, 0)),
        ],
        out_specs=pl.BlockSpec((bn, f), lambda i: (i, 0)),
        out_shape=jax.ShapeDtypeStruct((n, f), jnp.float32),
    )(agg_slabs, lin2_w, lin2_b.reshape(1, f), lin_w, lin_b.reshape(1, f))


def kernel(x, edge_index, edge_weight, edge_attr, nn_w1, nn_b1, nn_w2,
           nn_b2, lin1_w, lin2_w, lin2_b, lin_w, lin_b):
    n, f = x.shape
    e = edge_index.shape[1]
    half = f // 2

    be = 2048
    grain = max(NS * CHUNK * BODY, be)
    e_pad = ((e + grain - 1) // grain) * grain
    pad = e_pad - e

    src = edge_index[0]
    dst = edge_index[1]
    if pad:
        zi = jnp.zeros((pad,), jnp.int32)
        src = jnp.concatenate([src, zi])
        dst = jnp.concatenate([dst, jnp.full((pad,), n, jnp.int32)])

    h_slabs = _compute_h_slabs(x, lin1_w, bn=1000)
    w_slabs = _compute_w_slabs(edge_attr, edge_weight, nn_w1, nn_b1,
                               nn_w2, nn_b2, e_pad, be=be)

    chunks = e_pad // NS // CHUNK
    hflat = h_slabs.reshape(NC * n, half)
    w4d = w_slabs.reshape(NC * NS, chunks, CHUNK, half)
    src4d = jnp.stack([src, src + n]).reshape(NC, NS, chunks, CHUNK)
    dst3d = dst.reshape(NS, chunks, CHUNK)

    agg_slabs = _sc_aggregate(hflat, w4d, src4d, dst3d, n, e_pad, half)

    return _tail(agg_slabs, lin2_w, lin2_b, lin_w, lin_b, n, bn=1000)
